# Initial kernel scaffold; baseline (speedup 1.0000x reference)
import jax
import jax.numpy as jnp
from jax.experimental import pallas as pl


def kernel(text, audio, vision, edge_index, W_t, b_t, W_a, b_a, W_v, b_v, Wih_f, Whh_f, bih_f, bhh_f, Wih_b, Whh_b, bih_b, bhh_b, gc_W, gc_b, Ws1, Wd1, a1, Wr1, Ws2, Wd2, a2, Wr2, g2_W, g2_al, g2_ar, lin_W, lin_b):
    raise NotImplementedError("write your pallas kernel here")


# speedup vs baseline: 14.7023x; 14.7023x over previous
import functools

import jax
import jax.numpy as jnp
from jax import lax
from jax.experimental import pallas as pl
from jax.experimental.pallas import tpu as pltpu
from jax.experimental.pallas import tpu_sc as plsc

_SC_CORES = 2
_SC_SUBCORES = 16
_NW = _SC_CORES * _SC_SUBCORES


def _sc_mesh():
    return plsc.VectorSubcoreMesh(
        core_axis_name="c", subcore_axis_name="s",
        num_cores=_SC_CORES, num_subcores=_SC_SUBCORES)


def _sc_gather(table, idx):
    T, F0 = table.shape
    F = (F0 + 127) // 128 * 128
    if F != F0:
        table = jnp.pad(table, ((0, 0), (0, F - F0)))
    (E,) = idx.shape
    per_w = E // _NW
    EB = 120
    assert E % _NW == 0 and per_w % EB == 0
    n_iter = per_w // EB

    def body(table_hbm, idx_hbm, out_hbm, idx_v, rows_v, sem):
        cid = lax.axis_index("c")
        sid = lax.axis_index("s")
        base = (sid * _SC_CORES + cid) * per_w

        def step(i, carry):
            e0 = base + i * EB
            pltpu.sync_copy(idx_hbm.at[pl.ds(e0, EB)], idx_v)
            pltpu.async_copy(table_hbm.at[idx_v], rows_v, sem).wait()
            pltpu.sync_copy(rows_v, out_hbm.at[pl.ds(e0, EB)])
            return carry

        lax.fori_loop(0, n_iter, step, 0)

    out = pl.kernel(
        body,
        out_type=jax.ShapeDtypeStruct((E, F), jnp.float32),
        mesh=_sc_mesh(),
        scratch_types=[
            pltpu.VMEM((EB,), jnp.int32),
            pltpu.VMEM((EB, F), jnp.float32),
            pltpu.SemaphoreType.DMA,
        ],
    )(table, idx)
    return out[:, :F0] if F != F0 else out


def _sc_segment_sum(msgs, idx, n_out):
    E, F0 = msgs.shape
    n_pad = (n_out + 127) // 128 * 128
    if n_pad * F0 * 4 <= 6_300_000:
        F = F0
        fc = F
    else:
        F = (F0 + 127) // 128 * 128
        if F != F0:
            msgs = jnp.pad(msgs, ((0, 0), (0, F - F0)))
        fc = 128
    assert F % fc == 0 and fc % 16 == 0
    nchunks = F // fc
    per_s = E // _SC_SUBCORES
    EB = 120
    assert per_s % EB == 0
    n_iter = per_s // EB
    rows_per_s = n_pad // _SC_SUBCORES

    ZB = 16
    assert rows_per_s % ZB == 0

    def body(msg_hbm, idx_hbm, out_hbm, idx_v, msg_v, zero_v, acc_sh, sem):
        cid = lax.axis_index("c")
        sid = lax.axis_index("s")
        z16 = jnp.zeros((16,), jnp.float32)

        for r in range(ZB):
            for j in range(fc // 16):
                zero_v[r, pl.ds(j * 16, 16)] = z16

        for fci in range(nchunks):
            f0 = fci * fc

            @pl.when((fci % _SC_CORES) == cid)
            def _chunk():
                def zstep(z, carry):
                    pltpu.sync_copy(
                        zero_v,
                        acc_sh.at[pl.ds(sid * rows_per_s + z * ZB, ZB)])
                    return carry

                lax.fori_loop(0, rows_per_s // ZB, zstep, 0)
                plsc.subcore_barrier()

                def estep(i, carry):
                    e0 = sid * per_s + i * EB
                    pltpu.sync_copy(idx_hbm.at[pl.ds(e0, EB)], idx_v)
                    if nchunks == 1:
                        pltpu.sync_copy(msg_hbm.at[pl.ds(e0, EB)], msg_v)
                    else:
                        pltpu.sync_copy(
                            msg_hbm.at[pl.ds(e0, EB), pl.ds(f0, fc)], msg_v)
                    pltpu.sync_copy(msg_v, acc_sh.at[idx_v], add=True)
                    return carry

                lax.fori_loop(0, n_iter, estep, 0)
                plsc.subcore_barrier()
                if nchunks == 1:
                    pltpu.sync_copy(
                        acc_sh.at[pl.ds(sid * rows_per_s, rows_per_s)],
                        out_hbm.at[pl.ds(sid * rows_per_s, rows_per_s)])
                else:
                    pltpu.sync_copy(
                        acc_sh.at[pl.ds(sid * rows_per_s, rows_per_s)],
                        out_hbm.at[pl.ds(sid * rows_per_s, rows_per_s),
                                   pl.ds(f0, fc)])
                plsc.subcore_barrier()

    out = pl.kernel(
        body,
        out_type=jax.ShapeDtypeStruct((n_pad, F), jnp.float32),
        mesh=_sc_mesh(),
        scratch_types=[
            pltpu.VMEM((EB,), jnp.int32),
            pltpu.VMEM((EB, fc), jnp.float32),
            pltpu.VMEM((ZB, fc), jnp.float32),
            pltpu.VMEM_SHARED((n_pad, fc), jnp.float32),
            pltpu.SemaphoreType.DMA,
        ],
    )(msgs, idx)
    return out[:n_out, :F0] if (n_pad != n_out or F != F0) else out


_BN = 600
_BE = 1536


def _rows(blk, ncols):
    return pl.BlockSpec((blk, ncols), lambda i: (i, 0))


def _full(shape):
    return pl.BlockSpec(shape, lambda i: tuple(0 for _ in shape))


def _proj_body(t, a, v, wt, wa, wv, bt, ba, bv, ot, oa, ov):
    f32 = jnp.float32
    ot[...] = jnp.dot(t[...], wt[...], preferred_element_type=f32) + bt[...]
    oa[...] = jnp.dot(a[...], wa[...], preferred_element_type=f32) + ba[...]
    ov[...] = jnp.dot(v[...], wv[...], preferred_element_type=f32) + bv[...]


def _tc_proj(text, audio, vision, W_t, b_t, W_a, b_a, W_v, b_v):
    n = text.shape[0]
    grid = (n // _BN,)
    out = jax.ShapeDtypeStruct((n, 64), jnp.float32)
    t64, a64, v64 = pl.pallas_call(
        _proj_body,
        grid=grid,
        in_specs=[_rows(_BN, 1024), _rows(_BN, 512), _rows(_BN, 1024),
                  _full((1024, 64)), _full((512, 64)), _full((1024, 64)),
                  _full((1, 64)), _full((1, 64)), _full((1, 64))],
        out_specs=[_rows(_BN, 64)] * 3,
        out_shape=[out, out, out],
    )(text, audio, vision, W_t, W_a, W_v,
      b_t.reshape(1, 64), b_a.reshape(1, 64), b_v.reshape(1, 64))
    return jnp.concatenate([t64, a64, v64], axis=1)


def _lstm_body(xs, wii, wif, wig, wio, whi, whf, whg, who, bi, bf, bg, bo,
               wii2, wif2, wig2, wio2, whi2, whf2, whg2, who2,
               bi2, bf2, bg2, bo2, hf, hb):
    T = xs.shape[0]
    B = xs.shape[1]
    H = whi.shape[0]
    f32 = jnp.float32

    def run(dirn, wi4, wh4, b4, out):
        wi_i, wi_f, wi_g, wi_o = wi4
        wh_i, wh_f, wh_g, wh_o = wh4
        b_i, b_f, b_g, b_o = b4

        def step(t, carry):
            h, c = carry
            ti = t if dirn == 0 else (T - 1) - t
            xt = jnp.reshape(xs[pl.ds(ti, 1)], (B, xs.shape[2]))
            ig = jnp.dot(xt, wi_i[...], preferred_element_type=f32) \
                + jnp.dot(h, wh_i[...], preferred_element_type=f32) + b_i[...]
            fg = jnp.dot(xt, wi_f[...], preferred_element_type=f32) \
                + jnp.dot(h, wh_f[...], preferred_element_type=f32) + b_f[...]
            gg = jnp.dot(xt, wi_g[...], preferred_element_type=f32) \
                + jnp.dot(h, wh_g[...], preferred_element_type=f32) + b_g[...]
            og = jnp.dot(xt, wi_o[...], preferred_element_type=f32) \
                + jnp.dot(h, wh_o[...], preferred_element_type=f32) + b_o[...]
            c = jax.nn.sigmoid(fg) * c + jax.nn.sigmoid(ig) * jnp.tanh(gg)
            h = jax.nn.sigmoid(og) * jnp.tanh(c)
            out[pl.ds(ti, 1)] = jnp.reshape(h, (1, B, H))
            return (h, c)

        init = (jnp.zeros((B, H), f32), jnp.zeros((B, H), f32))
        lax.fori_loop(0, T, step, init)

    run(0, (wii, wif, wig, wio), (whi, whf, whg, who), (bi, bf, bg, bo), hf)
    run(1, (wii2, wif2, wig2, wio2), (whi2, whf2, whg2, who2),
        (bi2, bf2, bg2, bo2), hb)


def _tc_lstm(nf3, Wih_f, Whh_f, bih_f, bhh_f, Wih_b, Whh_b, bih_b, bhh_b):
    T, B, D = nf3.shape
    H = Whh_f.shape[1]

    def splits(Wih, Whh, bih, bhh):
        WiT = Wih.T
        WhT = Whh.T
        b = (bih + bhh).reshape(1, 4 * H)
        wi = [WiT[:, k * H:(k + 1) * H] for k in range(4)]
        wh = [WhT[:, k * H:(k + 1) * H] for k in range(4)]
        bs = [b[:, k * H:(k + 1) * H] for k in range(4)]
        return wi, wh, bs

    wi_f, wh_f, b_f = splits(Wih_f, Whh_f, bih_f, bhh_f)
    wi_b, wh_b, b_b = splits(Wih_b, Whh_b, bih_b, bhh_b)
    out = jax.ShapeDtypeStruct((T, B, H), jnp.float32)
    specs = ([_full((T, B, D))]
             + [_full((D, H))] * 4 + [_full((H, H))] * 4 + [_full((1, H))] * 4
             + [_full((D, H))] * 4 + [_full((H, H))] * 4 + [_full((1, H))] * 4)
    hf, hb = pl.pallas_call(
        _lstm_body,
        grid=(1,),
        in_specs=specs,
        out_specs=[_full((T, B, H))] * 2,
        out_shape=[out, out],
    )(nf3, *wi_f, *wh_f, *b_f, *wi_b, *wh_b, *b_b)
    return hf, hb


def _scale_body(x, deg, o):
    d = jnp.maximum(deg[:, 0:1], 1.0)
    o[...] = x[...] * lax.rsqrt(d)


def _tc_scale_by_deg(x, deg16):
    n, f = x.shape
    return pl.pallas_call(
        _scale_body,
        grid=(n // _BN,),
        in_specs=[_rows(_BN, f), _rows(_BN, 16)],
        out_specs=_rows(_BN, f),
        out_shape=jax.ShapeDtypeStruct((n, f), jnp.float32),
    )(x, deg16)


def _combine_body(agg, deg, stack, gcw, gcb, o):
    di = lax.rsqrt(jnp.maximum(deg[:, 0:1], 1.0))
    h1 = jnp.dot(agg[...] * di, gcw[...],
                 preferred_element_type=jnp.float32) + gcb[...]
    h = 0.5 * (stack[...] + h1)
    s = jnp.sum(jnp.abs(h), axis=1, keepdims=True)
    o[...] = h / jnp.maximum(s, 1e-12)


def _tc_combine(agg, deg16, stack, gc_W, gc_b):
    n, f = agg.shape
    return pl.pallas_call(
        _combine_body,
        grid=(n // _BN,),
        in_specs=[_rows(_BN, f), _rows(_BN, 16), _rows(_BN, f),
                  _full((f, f)), _full((1, f))],
        out_specs=_rows(_BN, f),
        out_shape=jax.ShapeDtypeStruct((n, f), jnp.float32),
    )(agg, deg16, stack, gc_W, gc_b.reshape(1, f))


def _cross_proj_body(h, w, al, ar, f3, el, er):
    f = jnp.dot(h[...], w[...], preferred_element_type=jnp.float32)
    f3[...] = f
    el[...] = jnp.dot(f, al[...], preferred_element_type=jnp.float32)
    er[...] = jnp.dot(f, ar[...], preferred_element_type=jnp.float32)


def _tc_cross_proj(h, g2_W, Al, Ar):
    n, k = h.shape
    f = g2_W.shape[1]
    return pl.pallas_call(
        _cross_proj_body,
        grid=(n // _BN,),
        in_specs=[_rows(_BN, k), _full((k, f)), _full((f, 16)),
                  _full((f, 16))],
        out_specs=[_rows(_BN, f), _rows(_BN, 16), _rows(_BN, 16)],
        out_shape=[jax.ShapeDtypeStruct((n, f), jnp.float32),
                   jax.ShapeDtypeStruct((n, 16), jnp.float32),
                   jax.ShapeDtypeStruct((n, 16), jnp.float32)],
    )(h, g2_W, Al, Ar)


def _leaky(x):
    return jnp.where(x >= 0, x, 0.2 * x)


def _exp_sum_body(a, b, o):
    o[...] = jnp.exp(_leaky(a[...] + b[...]))


def _tc_exp_logits_direct(el_s, er_d):
    e = el_s.shape[0]
    return pl.pallas_call(
        _exp_sum_body,
        grid=(e // _BE,),
        in_specs=[_rows(_BE, 16), _rows(_BE, 16)],
        out_specs=_rows(_BE, 16),
        out_shape=jax.ShapeDtypeStruct((e, 16), jnp.float32),
    )(el_s, er_d)


def _exp_blk_body(fs, fd, ab, o):
    e = _leaky(fs[...] + fd[...])
    o[...] = jnp.exp(jnp.dot(e, ab[...], preferred_element_type=jnp.float32))


def _tc_exp_logits_blk(fs_s, fd_d, Ablk):
    e, f = fs_s.shape
    return pl.pallas_call(
        _exp_blk_body,
        grid=(e // _BE,),
        in_specs=[_rows(_BE, f), _rows(_BE, f), _full((f, 16))],
        out_specs=_rows(_BE, 16),
        out_shape=jax.ShapeDtypeStruct((e, 16), jnp.float32),
    )(fs_s, fd_d, Ablk)


def _msg_body(ex, sg, fsrc, em, o):
    alpha = ex[...] / (sg[...] + 1e-9)
    o[...] = fsrc[...] * jnp.dot(alpha, em[...],
                                 preferred_element_type=jnp.float32)


def _tc_messages(ex, sg, fsrc, expand):
    e, f = fsrc.shape
    return pl.pallas_call(
        _msg_body,
        grid=(e // _BE,),
        in_specs=[_rows(_BE, 16), _rows(_BE, 16), _rows(_BE, f),
                  _full((16, f))],
        out_specs=_rows(_BE, f),
        out_shape=jax.ShapeDtypeStruct((e, f), jnp.float32),
    )(ex, sg, fsrc, expand)


def _proj3_body(x, ws, wd, wr, fs, fd, res):
    xv = x[...]
    fs[...] = jnp.dot(xv, ws[...], preferred_element_type=jnp.float32)
    fd[...] = jnp.dot(xv, wd[...], preferred_element_type=jnp.float32)
    res[...] = jnp.dot(xv, wr[...], preferred_element_type=jnp.float32)


def _tc_proj3(x, Ws, Wd, Wr):
    n, k = x.shape
    f = Ws.shape[1]
    out = jax.ShapeDtypeStruct((n, f), jnp.float32)
    return pl.pallas_call(
        _proj3_body,
        grid=(n // _BN,),
        in_specs=[_rows(_BN, k), _full((k, f)), _full((k, f)), _full((k, f))],
        out_specs=[_rows(_BN, f)] * 3,
        out_shape=[out, out, out],
    )(x, Ws, Wd, Wr)


def _proj3_relu_body(agg, res0, ws, wd, wr, fs, fd, res):
    xv = jnp.maximum(agg[...] + res0[...], 0.0)
    fs[...] = jnp.dot(xv, ws[...], preferred_element_type=jnp.float32)
    fd[...] = jnp.dot(xv, wd[...], preferred_element_type=jnp.float32)
    res[...] = jnp.dot(xv, wr[...], preferred_element_type=jnp.float32)


def _tc_proj3_relu(agg, res0, Ws, Wd, Wr):
    n, k = agg.shape
    f = Ws.shape[1]
    out = jax.ShapeDtypeStruct((n, f), jnp.float32)
    return pl.pallas_call(
        _proj3_relu_body,
        grid=(n // _BN,),
        in_specs=[_rows(_BN, k), _rows(_BN, k),
                  _full((k, f)), _full((k, f)), _full((k, f))],
        out_specs=[_rows(_BN, f)] * 3,
        out_shape=[out, out, out],
    )(agg, res0, Ws, Wd, Wr)


def _final_body(agg2, res2, nf, h3, w2, wn, w3, b, o):
    f32 = jnp.float32
    h2 = jnp.maximum(agg2[...] + res2[...], 0.0)
    h3r = jnp.maximum(h3[...], 0.0)
    o[...] = (jnp.dot(h2, w2[...], preferred_element_type=f32)
              + jnp.dot(nf[...], wn[...], preferred_element_type=f32)
              + jnp.dot(h3r, w3[...], preferred_element_type=f32) + b[...])


def _tc_final(agg2, res2, nf, h3, lin_W, lin_b):
    n = agg2.shape[0]
    osz = lin_W.shape[1]
    w2 = lin_W[0:64]
    wn = lin_W[64:72]
    w3 = lin_W[72:136]
    return pl.pallas_call(
        _final_body,
        grid=(n // _BN,),
        in_specs=[_rows(_BN, 64), _rows(_BN, 64), _rows(_BN, 8),
                  _rows(_BN, 64), _full((64, osz)), _full((8, osz)),
                  _full((64, osz)), _full((1, osz))],
        out_specs=_rows(_BN, osz),
        out_shape=jax.ShapeDtypeStruct((n, osz), jnp.float32),
    )(agg2, res2, nf, h3, w2, wn, w3, lin_b.reshape(1, osz))


def _block_diag(a):
    H, D = a.shape
    eye = jnp.eye(H, dtype=a.dtype)
    return (a[:, :, None] * eye[:, None, :]).reshape(H * D, H)


def _head_expand(H, D):
    return jnp.repeat(jnp.eye(H, dtype=jnp.float32), D, axis=1)


def _gatv2_layer(x, src, dst, Ws, Wd, a, Wr, heads, od, n):
    fs, fd, res = _tc_proj3(x, Ws, Wd, Wr)
    return _gatv2_edges(x, fs, fd, res, src, dst, a, heads, od, n)


def _gatv2_edges(x, fs, fd, res, src, dst, a, heads, od, n):
    F = heads * od
    fs_s = _sc_gather(fs, src)
    fd_d = _sc_gather(fd, dst)
    ex = _tc_exp_logits_blk(fs_s, fd_d, _block_diag(a))
    s = _sc_segment_sum(ex, dst, n)
    sg = _sc_gather(s, dst)
    msg = _tc_messages(ex, sg, fs_s, _head_expand(heads, od))
    agg = _sc_segment_sum(msg, dst, n)
    return agg, res


def kernel(text, audio, vision, edge_index, W_t, b_t, W_a, b_a, W_v, b_v,
           Wih_f, Whh_f, bih_f, bhh_f, Wih_b, Whh_b, bih_b, bhh_b,
           gc_W, gc_b, Ws1, Wd1, a1, Wr1, Ws2, Wd2, a2, Wr2,
           g2_W, g2_al, g2_ar, lin_W, lin_b):
    n = text.shape[0]
    src = edge_index[0]
    dst = edge_index[1]
    E = src.shape[0]

    stack = _tc_proj(text, audio, vision, W_t, b_t, W_a, b_a, W_v, b_v)

    nf3 = stack.reshape(-1, 120, 192).transpose(1, 0, 2)
    hf, hb = _tc_lstm(nf3, Wih_f, Whh_f, bih_f, bhh_f,
                      Wih_b, Whh_b, bih_b, bhh_b)
    new_feature = jnp.concatenate([hf, hb], axis=-1).transpose(1, 0, 2)
    new_feature = new_feature.reshape(-1, 8)

    ones16 = jnp.ones((E, 16), jnp.float32)
    deg_out16 = _sc_segment_sum(ones16, src, n)
    deg_in16 = _sc_segment_sum(ones16, dst, n)
    hds = _tc_scale_by_deg(stack, deg_out16)
    hsrc = _sc_gather(hds, src)
    agg = _sc_segment_sum(hsrc, dst, n)
    h = _tc_combine(agg, deg_in16, stack, gc_W, gc_b)

    f3, el3, er3 = _tc_cross_proj(h, g2_W, _block_diag(g2_al),
                                  _block_diag(g2_ar))
    el_s = _sc_gather(el3, src)
    er_d = _sc_gather(er3, dst)
    ex3 = _tc_exp_logits_direct(el_s, er_d)
    s3 = _sc_segment_sum(ex3, dst, n)
    s3g = _sc_gather(s3, dst)
    f3s = _sc_gather(f3, src)
    msg3 = _tc_messages(ex3, s3g, f3s, _head_expand(16, 4))
    h3 = _sc_segment_sum(msg3, dst, n)

    agg1, res1 = _gatv2_layer(h, src, dst, Ws1, Wd1, a1, Wr1, 16, 32, n)

    fs2, fd2, res2 = _tc_proj3_relu(agg1, res1, Ws2, Wd2, Wr2)
    agg2, res2b = _gatv2_edges(None, fs2, fd2, res2, src, dst, a2, 16, 4, n)

    return _tc_final(agg2, res2b, new_feature, h3, lin_W, lin_b)
